# baseline (speedup 1.0000x reference)
import functools

import jax
import jax.numpy as jnp
from jax import lax
from jax.experimental import pallas as pl
from jax.experimental.pallas import tpu as pltpu
from jax.experimental.pallas import tpu_sc as plsc

N = 10000
F = 128
E = 320000
NC = 2
NS = 16
NW = NC * NS
EPW = E // NW
K = 40
CH = EPW // K
IB = 5
NB = CH // IB
IBD = 25
NBD = CH // IBD
RPT = N // NS
DEG_PAD = 10240
DPT = DEG_PAD // NS
BR = 2000

_mesh = plsc.VectorSubcoreMesh(core_axis_name="c", subcore_axis_name="s")


@functools.partial(
    pl.kernel,
    out_type=jax.ShapeDtypeStruct((NW, DPT), jnp.float32),
    mesh=_mesh,
    scratch_types=[
        pltpu.VMEM((IBD, 2, K), jnp.int32),
        pltpu.VMEM((48,), jnp.float32),
        pltpu.VMEM((DPT,), jnp.float32),
        pltpu.VMEM_SHARED((DEG_PAD,), jnp.float32),
    ],
)
def _sc_deg(ed_hbm, out_hbm, eb, onesv, zb, acc):
    cid = lax.axis_index("c")
    sid = lax.axis_index("s")
    w = cid * NS + sid
    zero16 = jnp.zeros((16,), jnp.float32)
    one16 = jnp.ones((16,), jnp.float32)
    for i in range(DPT // 16):
        zb[pl.ds(i * 16, 16)] = zero16
    for i in range(48 // 16):
        onesv[pl.ds(i * 16, 16)] = one16
    pltpu.sync_copy(zb, acc.at[pl.ds(sid * DPT, DPT)])
    plsc.subcore_barrier()

    def blk(nb, c):
        pltpu.sync_copy(ed_hbm.at[w, pl.ds(nb * IBD, IBD)], eb)
        for i in range(IBD):
            pltpu.sync_copy(onesv.at[pl.ds(0, K)], acc.at[eb.at[i, 1]],
                            add=True)
        return c

    lax.fori_loop(0, NBD, blk, 0)
    plsc.subcore_barrier()
    pltpu.sync_copy(acc.at[pl.ds(sid * DPT, DPT)], out_hbm.at[w])


@functools.partial(
    pl.kernel,
    out_type=jax.ShapeDtypeStruct((NW, RPT, F), jnp.float32),
    mesh=_mesh,
    scratch_types=[
        pltpu.VMEM((IB, 2, K), jnp.int32),
        pltpu.VMEM((IB, 2, K), jnp.int32),
        pltpu.VMEM((K, F), jnp.float32),
        pltpu.VMEM((K, F), jnp.float32),
        pltpu.VMEM((K, F), jnp.float32),
        pltpu.VMEM((K, F), jnp.float32),
        pltpu.VMEM_SHARED((N, F), jnp.float32),
        pltpu.SemaphoreType.DMA,
        pltpu.SemaphoreType.DMA,
        pltpu.SemaphoreType.DMA,
        pltpu.SemaphoreType.DMA,
        pltpu.SemaphoreType.DMA,
        pltpu.SemaphoreType.DMA,
    ],
)
def _sc_edges(hs_hbm, ed_hbm, out_hbm,
              e0, e1, buf0, buf1, buf2, buf3, acc,
              semi0, semi1, semg0, semg1, semg2, semg3):
    cid = lax.axis_index("c")
    sid = lax.axis_index("s")
    w = cid * NS + sid
    zero16 = jnp.zeros((16,), jnp.float32)

    def zrow(i, c):
        for k2 in range(F // 16):
            buf0[i, pl.ds(k2 * 16, 16)] = zero16
        return c

    lax.fori_loop(0, K, zrow, 0)
    row0 = sid * RPT
    for r in range(RPT // K):
        pltpu.sync_copy(buf0, acc.at[pl.ds(row0 + r * K, K)])
    rem = RPT % K
    if rem:
        pltpu.sync_copy(buf0.at[pl.ds(0, rem)],
                        acc.at[pl.ds(row0 + (RPT // K) * K, rem)])
    plsc.subcore_barrier()

    gbufs = (buf0, buf1, buf2, buf3)
    gsems = (semg0, semg1, semg2, semg3)
    esl = (e0, e1)
    isems = (semi0, semi1)

    pltpu.sync_copy(ed_hbm.at[w, pl.ds(0, IB)], e0)
    pltpu.async_copy(ed_hbm.at[w, pl.ds(IB, IB)], e1, semi1)
    for i in range(3):
        pltpu.async_copy(hs_hbm.at[e0.at[i, 0]], gbufs[i], gsems[i])

    def block(nb, v):
        cur, nxt = esl[v % 2], esl[(v + 1) % 2]
        for i in range(IB):
            gb, gs = gbufs[(v * IB + i) % 4], gsems[(v * IB + i) % 4]
            pb, ps = gbufs[(v * IB + i + 3) % 4], gsems[(v * IB + i + 3) % 4]
            if i < IB - 3:
                pltpu.async_copy(hs_hbm.at[cur.at[i + 3, 0]], pb, ps)
            elif i == IB - 3:

                @pl.when(nb + 1 < NB)
                def _(pb=pb, ps=ps):
                    pltpu.make_async_copy(
                        ed_hbm.at[w, pl.ds((nb + 1) * IB, IB)], nxt,
                        isems[(v + 1) % 2]).wait()
                    pltpu.async_copy(hs_hbm.at[nxt.at[0, 0]], pb, ps)
            else:

                @pl.when(nb + 1 < NB)
                def _(i=i, pb=pb, ps=ps):
                    pltpu.async_copy(hs_hbm.at[nxt.at[i - (IB - 3), 0]],
                                     pb, ps)
            pltpu.make_async_copy(hs_hbm.at[cur.at[i, 0]], gb, gs).wait()
            pltpu.sync_copy(gb, acc.at[cur.at[i, 1]], add=True)

        @pl.when(nb + 2 < NB)
        def _():
            pltpu.async_copy(ed_hbm.at[w, pl.ds((nb + 2) * IB, IB)], cur,
                             isems[v % 2])

    def step(nb, c):
        for v in range(4):

            @pl.when(nb % 4 == v)
            def _(v=v):
                block(nb, v)
        return c

    lax.fori_loop(0, NB, step, 0)
    plsc.subcore_barrier()
    pltpu.sync_copy(acc.at[pl.ds(row0, RPT)], out_hbm.at[w])


def _dis(deg_blk):
    return lax.rsqrt(1.0 + deg_blk[:, 0] + deg_blk[:, 1])


def _tc_in_body(x_ref, w_ref, deg_ref, o_ref):
    dis = _dis(deg_ref[...])
    h = jnp.dot(x_ref[...], w_ref[...], precision=lax.Precision.HIGHEST,
                preferred_element_type=jnp.float32)
    o_ref[...] = h * dis[:, None]


def _tc_mid_body(p_ref, hs_ref, deg_ref, b_ref, w_ref, o_ref):
    dis = _dis(deg_ref[...])
    u = (p_ref[0] + p_ref[1] + hs_ref[...]) * dis[:, None] + b_ref[...]
    u = jnp.maximum(u, 0.0)
    h = jnp.dot(u, w_ref[...], precision=lax.Precision.HIGHEST,
                preferred_element_type=jnp.float32)
    o_ref[...] = h * dis[:, None]


def _tc_out_body(p_ref, hs_ref, deg_ref, b_ref, o_ref):
    dis = _dis(deg_ref[...])
    o_ref[...] = (p_ref[0] + p_ref[1] + hs_ref[...]) * dis[:, None] + b_ref[...]


_row_spec = pl.BlockSpec((BR, F), lambda i: (i, 0))
_p_spec = pl.BlockSpec((NC, BR, F), lambda i: (0, i, 0))
_deg_spec = pl.BlockSpec((BR, 2), lambda i: (i, 0))
_w_spec = pl.BlockSpec((F, F), lambda i: (0, 0))
_b_spec = pl.BlockSpec((1, F), lambda i: (0, 0))
_out_sds = jax.ShapeDtypeStruct((N, F), jnp.float32)

_tc_in = pl.pallas_call(
    _tc_in_body, grid=(N // BR,),
    in_specs=[_row_spec, _w_spec, _deg_spec],
    out_specs=_row_spec, out_shape=_out_sds)

_tc_mid = pl.pallas_call(
    _tc_mid_body, grid=(N // BR,),
    in_specs=[_p_spec, _row_spec, _deg_spec, _b_spec, _w_spec],
    out_specs=_row_spec, out_shape=_out_sds)

_tc_out = pl.pallas_call(
    _tc_out_body, grid=(N // BR,),
    in_specs=[_p_spec, _row_spec, _deg_spec, _b_spec],
    out_specs=_row_spec, out_shape=_out_sds)


def kernel(x, edge_index, W1, b1, W2, b2, W3, b3):
    ei = edge_index.astype(jnp.int32)
    ed = jnp.stack(
        [ei[0].reshape(NW, CH, K), ei[1].reshape(NW, CH, K)], axis=2)
    deg2 = _sc_deg(ed).reshape(NC, DEG_PAD)
    degT = deg2[:, :N].T
    b1r, b2r, b3r = (b.reshape(1, F) for b in (b1, b2, b3))

    hs1 = _tc_in(x, W1, degT)
    p1 = _sc_edges(hs1, ed).reshape(NC, N, F)
    hs2 = _tc_mid(p1, hs1, degT, b1r, W2)
    p2 = _sc_edges(hs2, ed).reshape(NC, N, F)
    hs3 = _tc_mid(p2, hs2, degT, b2r, W3)
    p3 = _sc_edges(hs3, ed).reshape(NC, N, F)
    return _tc_out(p3, hs3, degT, b3r)


# speedup vs baseline: 1.0062x; 1.0062x over previous
import functools

import jax
import jax.numpy as jnp
from jax import lax
from jax.experimental import pallas as pl
from jax.experimental.pallas import tpu as pltpu
from jax.experimental.pallas import tpu_sc as plsc

N = 10000
F = 128
E = 320000
NC = 2
NS = 16
NW = NC * NS
EPW = E // NW
K = 40
CH = EPW // K
IB = 5
NB = CH // IB
IBD = 25
NBD = CH // IBD
RPT = N // NS
DEG_PAD = 10240
DPT = DEG_PAD // NS
BR = 2000

_mesh = plsc.VectorSubcoreMesh(core_axis_name="c", subcore_axis_name="s")


@functools.partial(
    pl.kernel,
    out_type=jax.ShapeDtypeStruct((NW, DPT), jnp.float32),
    mesh=_mesh,
    scratch_types=[
        pltpu.VMEM((IBD, 2, K), jnp.int32),
        pltpu.VMEM((48,), jnp.float32),
        pltpu.VMEM((DPT,), jnp.float32),
        pltpu.VMEM_SHARED((DEG_PAD,), jnp.float32),
    ],
)
def _sc_deg(ed_hbm, out_hbm, eb, onesv, zb, acc):
    cid = lax.axis_index("c")
    sid = lax.axis_index("s")
    w = cid * NS + sid
    zero16 = jnp.zeros((16,), jnp.float32)
    one16 = jnp.ones((16,), jnp.float32)
    for i in range(DPT // 16):
        zb[pl.ds(i * 16, 16)] = zero16
    for i in range(48 // 16):
        onesv[pl.ds(i * 16, 16)] = one16
    pltpu.sync_copy(zb, acc.at[pl.ds(sid * DPT, DPT)])
    plsc.subcore_barrier()

    def blk(nb, c):
        pltpu.sync_copy(ed_hbm.at[w, pl.ds(nb * IBD, IBD)], eb)
        for i in range(IBD):
            pltpu.sync_copy(onesv.at[pl.ds(0, K)], acc.at[eb.at[i, 1]],
                            add=True)
        return c

    lax.fori_loop(0, NBD, blk, 0)
    plsc.subcore_barrier()
    pltpu.sync_copy(acc.at[pl.ds(sid * DPT, DPT)], out_hbm.at[w])


@functools.partial(
    pl.kernel,
    out_type=jax.ShapeDtypeStruct((NW, RPT, F), jnp.float32),
    mesh=_mesh,
    scratch_types=[
        pltpu.VMEM((IB, 2, K), jnp.int32),
        pltpu.VMEM((IB, 2, K), jnp.int32),
        pltpu.VMEM((K, F), jnp.float32),
        pltpu.VMEM((K, F), jnp.float32),
        pltpu.VMEM((K, F), jnp.float32),
        pltpu.VMEM((K, F), jnp.float32),
        pltpu.VMEM_SHARED((N, F), jnp.float32),
        pltpu.SemaphoreType.DMA,
        pltpu.SemaphoreType.DMA,
        pltpu.SemaphoreType.DMA,
        pltpu.SemaphoreType.DMA,
        pltpu.SemaphoreType.DMA,
        pltpu.SemaphoreType.DMA,
        pltpu.SemaphoreType.DMA,
        pltpu.SemaphoreType.DMA,
        pltpu.SemaphoreType.DMA,
        pltpu.SemaphoreType.DMA,
    ],
)
def _sc_edges(hs_hbm, ed_hbm, out_hbm,
              e0, e1, buf0, buf1, buf2, buf3, acc,
              semi0, semi1, semg0, semg1, semg2, semg3,
              semr0, semr1, semr2, semr3):
    cid = lax.axis_index("c")
    sid = lax.axis_index("s")
    w = cid * NS + sid
    zero16 = jnp.zeros((16,), jnp.float32)

    def zrow(i, c):
        for k2 in range(F // 16):
            buf0[i, pl.ds(k2 * 16, 16)] = zero16
        return c

    lax.fori_loop(0, K, zrow, 0)
    row0 = sid * RPT
    for r in range(RPT // K):
        pltpu.sync_copy(buf0, acc.at[pl.ds(row0 + r * K, K)])
    rem = RPT % K
    if rem:
        pltpu.sync_copy(buf0.at[pl.ds(0, rem)],
                        acc.at[pl.ds(row0 + (RPT // K) * K, rem)])
    plsc.subcore_barrier()

    gbufs = (buf0, buf1, buf2, buf3)
    gsems = (semg0, semg1, semg2, semg3)
    esl = (e0, e1)
    isems = (semi0, semi1)

    pltpu.sync_copy(ed_hbm.at[w, pl.ds(0, IB)], e0)
    pltpu.async_copy(ed_hbm.at[w, pl.ds(IB, IB)], e1, semi1)
    for i in range(3):
        pltpu.async_copy(hs_hbm.at[e0.at[i, 0]], gbufs[i], gsems[i])

    rsems = (semr0, semr1, semr2, semr3)

    def block(nb, v):
        cur, nxt = esl[v % 2], esl[(v + 1) % 2]
        for i in range(IB):
            m = (v * IB + i) % 4
            gb, gs = gbufs[m], gsems[m]
            pm = (m + 3) % 4
            pb, ps = gbufs[pm], gsems[pm]
            if i == 0:

                @pl.when(nb > 0)
                def _(pm=pm):
                    pltpu.make_async_copy(
                        gbufs[pm], acc.at[pl.ds(row0, K)],
                        rsems[pm]).wait()
            else:
                pltpu.make_async_copy(gbufs[pm], acc.at[pl.ds(row0, K)],
                                      rsems[pm]).wait()
            if i < IB - 3:
                pltpu.async_copy(hs_hbm.at[cur.at[i + 3, 0]], pb, ps)
            elif i == IB - 3:

                @pl.when(nb + 1 < NB)
                def _(pb=pb, ps=ps):
                    pltpu.make_async_copy(
                        ed_hbm.at[w, pl.ds((nb + 1) * IB, IB)], nxt,
                        isems[(v + 1) % 2]).wait()
                    pltpu.async_copy(hs_hbm.at[nxt.at[0, 0]], pb, ps)
            else:

                @pl.when(nb + 1 < NB)
                def _(i=i, pb=pb, ps=ps):
                    pltpu.async_copy(hs_hbm.at[nxt.at[i - (IB - 3), 0]],
                                     pb, ps)
            pltpu.make_async_copy(hs_hbm.at[cur.at[i, 0]], gb, gs).wait()
            pltpu.async_copy(gb, acc.at[cur.at[i, 1]], rsems[m], add=True)

        @pl.when(nb + 2 < NB)
        def _():
            pltpu.async_copy(ed_hbm.at[w, pl.ds((nb + 2) * IB, IB)], cur,
                             isems[v % 2])

    def step(nb, c):
        for v in range(4):

            @pl.when(nb % 4 == v)
            def _(v=v):
                block(nb, v)
        return c

    lax.fori_loop(0, NB, step, 0)
    pltpu.make_async_copy(gbufs[(CH - 1) % 4], acc.at[pl.ds(row0, K)],
                          rsems[(CH - 1) % 4]).wait()
    plsc.subcore_barrier()
    pltpu.sync_copy(acc.at[pl.ds(row0, RPT)], out_hbm.at[w])


def _dis(deg_blk):
    return lax.rsqrt(1.0 + deg_blk[:, 0] + deg_blk[:, 1])


def _tc_in_body(x_ref, w_ref, deg_ref, o_ref):
    dis = _dis(deg_ref[...])
    h = jnp.dot(x_ref[...], w_ref[...], precision=lax.Precision.HIGHEST,
                preferred_element_type=jnp.float32)
    o_ref[...] = h * dis[:, None]


def _tc_mid_body(p_ref, hs_ref, deg_ref, b_ref, w_ref, o_ref):
    dis = _dis(deg_ref[...])
    u = (p_ref[0] + p_ref[1] + hs_ref[...]) * dis[:, None] + b_ref[...]
    u = jnp.maximum(u, 0.0)
    h = jnp.dot(u, w_ref[...], precision=lax.Precision.HIGHEST,
                preferred_element_type=jnp.float32)
    o_ref[...] = h * dis[:, None]


def _tc_out_body(p_ref, hs_ref, deg_ref, b_ref, o_ref):
    dis = _dis(deg_ref[...])
    o_ref[...] = (p_ref[0] + p_ref[1] + hs_ref[...]) * dis[:, None] + b_ref[...]


_row_spec = pl.BlockSpec((BR, F), lambda i: (i, 0))
_p_spec = pl.BlockSpec((NC, BR, F), lambda i: (0, i, 0))
_deg_spec = pl.BlockSpec((BR, 2), lambda i: (i, 0))
_w_spec = pl.BlockSpec((F, F), lambda i: (0, 0))
_b_spec = pl.BlockSpec((1, F), lambda i: (0, 0))
_out_sds = jax.ShapeDtypeStruct((N, F), jnp.float32)

_tc_in = pl.pallas_call(
    _tc_in_body, grid=(N // BR,),
    in_specs=[_row_spec, _w_spec, _deg_spec],
    out_specs=_row_spec, out_shape=_out_sds)

_tc_mid = pl.pallas_call(
    _tc_mid_body, grid=(N // BR,),
    in_specs=[_p_spec, _row_spec, _deg_spec, _b_spec, _w_spec],
    out_specs=_row_spec, out_shape=_out_sds)

_tc_out = pl.pallas_call(
    _tc_out_body, grid=(N // BR,),
    in_specs=[_p_spec, _row_spec, _deg_spec, _b_spec],
    out_specs=_row_spec, out_shape=_out_sds)


def kernel(x, edge_index, W1, b1, W2, b2, W3, b3):
    ei = edge_index.astype(jnp.int32)
    ed = jnp.stack(
        [ei[0].reshape(NW, CH, K), ei[1].reshape(NW, CH, K)], axis=2)
    deg2 = _sc_deg(ed).reshape(NC, DEG_PAD)
    degT = deg2[:, :N].T
    b1r, b2r, b3r = (b.reshape(1, F) for b in (b1, b2, b3))

    hs1 = _tc_in(x, W1, degT)
    p1 = _sc_edges(hs1, ed).reshape(NC, N, F)
    hs2 = _tc_mid(p1, hs1, degT, b1r, W2)
    p2 = _sc_edges(hs2, ed).reshape(NC, N, F)
    hs3 = _tc_mid(p2, hs2, degT, b2r, W3)
    p3 = _sc_edges(hs3, ed).reshape(NC, N, F)
    return _tc_out(p3, hs3, degT, b3r)


# speedup vs baseline: 1.0546x; 1.0481x over previous
import functools

import jax
import jax.numpy as jnp
from jax import lax
from jax.experimental import pallas as pl
from jax.experimental.pallas import tpu as pltpu
from jax.experimental.pallas import tpu_sc as plsc

N = 10000
F = 128
E = 320000
NC = 2
NS = 16
NW = NC * NS
EPW = E // NW
K = 40
CH = EPW // K
IB = 5
NB = CH // IB
RPT = N // NS
DEG_PAD = 10240
DPT = DEG_PAD // NS
BR = 2000

_mesh = plsc.VectorSubcoreMesh(core_axis_name="c", subcore_axis_name="s")


@functools.partial(
    pl.kernel,
    out_type=jax.ShapeDtypeStruct((NW, DPT), jnp.float32),
    mesh=_mesh,
    scratch_types=[
        pltpu.VMEM((CH, 2, K), jnp.int32),
        pltpu.VMEM((48,), jnp.float32),
        pltpu.VMEM((DPT,), jnp.float32),
        pltpu.VMEM_SHARED((DEG_PAD,), jnp.float32),
        pltpu.SemaphoreType.DMA,
    ],
)
def _sc_deg(ed_hbm, out_hbm, eb, onesv, zb, acc, dsem):
    cid = lax.axis_index("c")
    sid = lax.axis_index("s")
    w = cid * NS + sid
    zero16 = jnp.zeros((16,), jnp.float32)
    one16 = jnp.ones((16,), jnp.float32)
    for i in range(DPT // 16):
        zb[pl.ds(i * 16, 16)] = zero16
    for i in range(48 // 16):
        onesv[pl.ds(i * 16, 16)] = one16
    pltpu.sync_copy(zb, acc.at[pl.ds(sid * DPT, DPT)])
    plsc.subcore_barrier()

    pltpu.sync_copy(ed_hbm.at[w], eb)

    def blk(j, c):
        pltpu.async_copy(onesv.at[pl.ds(0, K)], acc.at[eb.at[j, 1]], dsem,
                         add=True)
        return c

    lax.fori_loop(0, CH, blk, 0)

    def drain(j, c):
        pltpu.make_async_copy(onesv.at[pl.ds(0, K)], acc.at[pl.ds(0, K)],
                              dsem).wait()
        return c

    lax.fori_loop(0, CH, drain, 0)
    plsc.subcore_barrier()
    pltpu.sync_copy(acc.at[pl.ds(sid * DPT, DPT)], out_hbm.at[w])


@functools.partial(
    pl.kernel,
    out_type=jax.ShapeDtypeStruct((NW, RPT, F), jnp.float32),
    mesh=_mesh,
    scratch_types=[
        pltpu.VMEM((IB, 2, K), jnp.int32),
        pltpu.VMEM((IB, 2, K), jnp.int32),
        pltpu.VMEM((K, F), jnp.float32),
        pltpu.VMEM((K, F), jnp.float32),
        pltpu.VMEM((K, F), jnp.float32),
        pltpu.VMEM((K, F), jnp.float32),
        pltpu.VMEM_SHARED((N, F), jnp.float32),
        pltpu.SemaphoreType.DMA,
        pltpu.SemaphoreType.DMA,
        pltpu.SemaphoreType.DMA,
        pltpu.SemaphoreType.DMA,
        pltpu.SemaphoreType.DMA,
        pltpu.SemaphoreType.DMA,
        pltpu.SemaphoreType.DMA,
        pltpu.SemaphoreType.DMA,
        pltpu.SemaphoreType.DMA,
        pltpu.SemaphoreType.DMA,
    ],
)
def _sc_edges(hs_hbm, ed_hbm, out_hbm,
              e0, e1, buf0, buf1, buf2, buf3, acc,
              semi0, semi1, semg0, semg1, semg2, semg3,
              semr0, semr1, semr2, semr3):
    cid = lax.axis_index("c")
    sid = lax.axis_index("s")
    w = cid * NS + sid
    zero16 = jnp.zeros((16,), jnp.float32)

    def zrow(i, c):
        for k2 in range(F // 16):
            buf0[i, pl.ds(k2 * 16, 16)] = zero16
        return c

    lax.fori_loop(0, K, zrow, 0)
    row0 = sid * RPT
    for r in range(RPT // K):
        pltpu.sync_copy(buf0, acc.at[pl.ds(row0 + r * K, K)])
    rem = RPT % K
    if rem:
        pltpu.sync_copy(buf0.at[pl.ds(0, rem)],
                        acc.at[pl.ds(row0 + (RPT // K) * K, rem)])
    plsc.subcore_barrier()

    gbufs = (buf0, buf1, buf2, buf3)
    gsems = (semg0, semg1, semg2, semg3)
    esl = (e0, e1)
    isems = (semi0, semi1)

    pltpu.sync_copy(ed_hbm.at[w, pl.ds(0, IB)], e0)
    pltpu.async_copy(ed_hbm.at[w, pl.ds(IB, IB)], e1, semi1)
    for i in range(3):
        pltpu.async_copy(hs_hbm.at[e0.at[i, 0]], gbufs[i], gsems[i])

    rsems = (semr0, semr1, semr2, semr3)

    def block(nb, v):
        cur, nxt = esl[v % 2], esl[(v + 1) % 2]
        for i in range(IB):
            m = (v * IB + i) % 4
            gb, gs = gbufs[m], gsems[m]
            pm = (m + 3) % 4
            pb, ps = gbufs[pm], gsems[pm]
            if i == 0:

                @pl.when(nb > 0)
                def _(pm=pm):
                    pltpu.make_async_copy(
                        gbufs[pm], acc.at[pl.ds(row0, K)],
                        rsems[pm]).wait()
            else:
                pltpu.make_async_copy(gbufs[pm], acc.at[pl.ds(row0, K)],
                                      rsems[pm]).wait()
            if i < IB - 3:
                pltpu.async_copy(hs_hbm.at[cur.at[i + 3, 0]], pb, ps)
            elif i == IB - 3:

                @pl.when(nb + 1 < NB)
                def _(pb=pb, ps=ps):
                    pltpu.make_async_copy(
                        ed_hbm.at[w, pl.ds((nb + 1) * IB, IB)], nxt,
                        isems[(v + 1) % 2]).wait()
                    pltpu.async_copy(hs_hbm.at[nxt.at[0, 0]], pb, ps)
            else:

                @pl.when(nb + 1 < NB)
                def _(i=i, pb=pb, ps=ps):
                    pltpu.async_copy(hs_hbm.at[nxt.at[i - (IB - 3), 0]],
                                     pb, ps)
            pltpu.make_async_copy(hs_hbm.at[cur.at[i, 0]], gb, gs).wait()
            pltpu.async_copy(gb, acc.at[cur.at[i, 1]], rsems[m], add=True)

        @pl.when(nb + 2 < NB)
        def _():
            pltpu.async_copy(ed_hbm.at[w, pl.ds((nb + 2) * IB, IB)], cur,
                             isems[v % 2])

    def step(nb, c):
        for v in range(4):

            @pl.when(nb % 4 == v)
            def _(v=v):
                block(nb, v)
        return c

    lax.fori_loop(0, NB, step, 0)
    pltpu.make_async_copy(gbufs[(NB * IB - 1) % 4], acc.at[pl.ds(row0, K)],
                          rsems[(NB * IB - 1) % 4]).wait()
    plsc.subcore_barrier()
    pltpu.sync_copy(acc.at[pl.ds(row0, RPT)], out_hbm.at[w])


def _dis(deg_blk):
    return lax.rsqrt(1.0 + deg_blk[:, 0] + deg_blk[:, 1])


def _tc_in_body(x_ref, w_ref, deg_ref, o_ref):
    dis = _dis(deg_ref[...])
    h = jnp.dot(x_ref[...], w_ref[...], precision=lax.Precision.HIGHEST,
                preferred_element_type=jnp.float32)
    o_ref[...] = h * dis[:, None]


def _tc_mid_body(p_ref, hs_ref, deg_ref, b_ref, w_ref, o_ref):
    dis = _dis(deg_ref[...])
    u = (p_ref[0] + p_ref[1] + hs_ref[...]) * dis[:, None] + b_ref[...]
    u = jnp.maximum(u, 0.0)
    h = jnp.dot(u, w_ref[...], precision=lax.Precision.HIGHEST,
                preferred_element_type=jnp.float32)
    o_ref[...] = h * dis[:, None]


def _tc_out_body(p_ref, hs_ref, deg_ref, b_ref, o_ref):
    dis = _dis(deg_ref[...])
    o_ref[...] = (p_ref[0] + p_ref[1] + hs_ref[...]) * dis[:, None] + b_ref[...]


_row_spec = pl.BlockSpec((BR, F), lambda i: (i, 0))
_p_spec = pl.BlockSpec((NC, BR, F), lambda i: (0, i, 0))
_deg_spec = pl.BlockSpec((BR, 2), lambda i: (i, 0))
_w_spec = pl.BlockSpec((F, F), lambda i: (0, 0))
_b_spec = pl.BlockSpec((1, F), lambda i: (0, 0))
_out_sds = jax.ShapeDtypeStruct((N, F), jnp.float32)

_tc_in = pl.pallas_call(
    _tc_in_body, grid=(N // BR,),
    in_specs=[_row_spec, _w_spec, _deg_spec],
    out_specs=_row_spec, out_shape=_out_sds)

_tc_mid = pl.pallas_call(
    _tc_mid_body, grid=(N // BR,),
    in_specs=[_p_spec, _row_spec, _deg_spec, _b_spec, _w_spec],
    out_specs=_row_spec, out_shape=_out_sds)

_tc_out = pl.pallas_call(
    _tc_out_body, grid=(N // BR,),
    in_specs=[_p_spec, _row_spec, _deg_spec, _b_spec],
    out_specs=_row_spec, out_shape=_out_sds)


def kernel(x, edge_index, W1, b1, W2, b2, W3, b3):
    ei = edge_index.astype(jnp.int32)
    ed = jnp.stack(
        [ei[0].reshape(NW, CH, K), ei[1].reshape(NW, CH, K)], axis=2)
    deg2 = _sc_deg(ed).reshape(NC, DEG_PAD)
    degT = deg2[:, :N].T
    b1r, b2r, b3r = (b.reshape(1, F) for b in (b1, b2, b3))

    hs1 = _tc_in(x, W1, degT)
    p1 = _sc_edges(hs1, ed).reshape(NC, N, F)
    hs2 = _tc_mid(p1, hs1, degT, b1r, W2)
    p2 = _sc_edges(hs2, ed).reshape(NC, N, F)
    hs3 = _tc_mid(p2, hs2, degT, b2r, W3)
    p3 = _sc_edges(hs3, ed).reshape(NC, N, F)
    return _tc_out(p3, hs3, degT, b3r)


# speedup vs baseline: 1.0582x; 1.0034x over previous
import functools

import jax
import jax.numpy as jnp
from jax import lax
from jax.experimental import pallas as pl
from jax.experimental.pallas import tpu as pltpu
from jax.experimental.pallas import tpu_sc as plsc

N = 10000
F = 128
E = 320000
NC = 2
NS = 16
NW = NC * NS
EPW = E // NW
K = 40
CH = EPW // K
IB = 5
NB = CH // IB
RPT = N // NS
DEG_PAD = 10240
DPT = DEG_PAD // NS
BR = 2000

_mesh = plsc.VectorSubcoreMesh(core_axis_name="c", subcore_axis_name="s")


@functools.partial(
    pl.kernel,
    out_type=jax.ShapeDtypeStruct((NW, DPT), jnp.float32),
    mesh=_mesh,
    scratch_types=[
        pltpu.VMEM((CH, 2, K), jnp.int32),
        pltpu.VMEM((48,), jnp.float32),
        pltpu.VMEM((DPT,), jnp.float32),
        pltpu.VMEM_SHARED((DEG_PAD,), jnp.float32),
        pltpu.SemaphoreType.DMA,
    ],
)
def _sc_deg(ed_hbm, out_hbm, eb, onesv, zb, acc, dsem):
    cid = lax.axis_index("c")
    sid = lax.axis_index("s")
    w = cid * NS + sid
    zero16 = jnp.zeros((16,), jnp.float32)
    one16 = jnp.ones((16,), jnp.float32)
    for i in range(DPT // 16):
        zb[pl.ds(i * 16, 16)] = zero16
    for i in range(48 // 16):
        onesv[pl.ds(i * 16, 16)] = one16
    pltpu.sync_copy(zb, acc.at[pl.ds(sid * DPT, DPT)])
    plsc.subcore_barrier()

    pltpu.sync_copy(ed_hbm.at[w], eb)

    def blk(j, c):
        pltpu.async_copy(onesv.at[pl.ds(0, K)], acc.at[eb.at[j, 1]], dsem,
                         add=True)
        return c

    lax.fori_loop(0, CH, blk, 0)

    def drain(j, c):
        pltpu.make_async_copy(onesv.at[pl.ds(0, K)], acc.at[pl.ds(0, K)],
                              dsem).wait()
        return c

    lax.fori_loop(0, CH, drain, 0)
    plsc.subcore_barrier()
    pltpu.sync_copy(acc.at[pl.ds(sid * DPT, DPT)], out_hbm.at[w])


@functools.partial(
    pl.kernel,
    out_type=jax.ShapeDtypeStruct((NW, RPT, F), jnp.float32),
    mesh=_mesh,
    scratch_types=[
        pltpu.VMEM((IB, 2, K), jnp.int32),
        pltpu.VMEM((IB, 2, K), jnp.int32),
        pltpu.VMEM((K, F), jnp.float32),
        pltpu.VMEM((K, F), jnp.float32),
        pltpu.VMEM((K, F), jnp.float32),
        pltpu.VMEM((K, F), jnp.float32),
        pltpu.VMEM_SHARED((N, F), jnp.float32),
        pltpu.SemaphoreType.DMA,
        pltpu.SemaphoreType.DMA,
        pltpu.SemaphoreType.DMA,
        pltpu.SemaphoreType.DMA,
        pltpu.SemaphoreType.DMA,
        pltpu.SemaphoreType.DMA,
        pltpu.SemaphoreType.DMA,
        pltpu.SemaphoreType.DMA,
        pltpu.SemaphoreType.DMA,
        pltpu.SemaphoreType.DMA,
    ],
)
def _sc_edges(hs_hbm, ed_hbm, out_hbm,
              e0, e1, buf0, buf1, buf2, buf3, acc,
              semi0, semi1, semg0, semg1, semg2, semg3,
              semr0, semr1, semr2, semr3):
    cid = lax.axis_index("c")
    sid = lax.axis_index("s")
    w = cid * NS + sid
    zero16 = jnp.zeros((16,), jnp.float32)

    def zrow(i, c):
        for k2 in range(F // 16):
            buf0[i, pl.ds(k2 * 16, 16)] = zero16
        return c

    lax.fori_loop(0, K, zrow, 0)
    row0 = sid * RPT
    for r in range(RPT // K):
        pltpu.async_copy(buf0, acc.at[pl.ds(row0 + r * K, K)], semi0)
    rem = RPT % K
    if rem:
        pltpu.async_copy(buf0.at[pl.ds(0, rem)],
                         acc.at[pl.ds(row0 + (RPT // K) * K, rem)], semi0)
    for r in range(RPT // K):
        pltpu.make_async_copy(buf0, acc.at[pl.ds(row0, K)], semi0).wait()
    if rem:
        pltpu.make_async_copy(buf0.at[pl.ds(0, rem)],
                              acc.at[pl.ds(row0, rem)], semi0).wait()
    plsc.subcore_barrier()

    gbufs = (buf0, buf1, buf2, buf3)
    gsems = (semg0, semg1, semg2, semg3)
    esl = (e0, e1)
    isems = (semi0, semi1)

    pltpu.sync_copy(ed_hbm.at[w, pl.ds(0, IB)], e0)
    pltpu.async_copy(ed_hbm.at[w, pl.ds(IB, IB)], e1, semi1)
    for i in range(3):
        pltpu.async_copy(hs_hbm.at[e0.at[i, 0]], gbufs[i], gsems[i])

    rsems = (semr0, semr1, semr2, semr3)

    def block(nb, v):
        cur, nxt = esl[v % 2], esl[(v + 1) % 2]
        for i in range(IB):
            m = (v * IB + i) % 4
            gb, gs = gbufs[m], gsems[m]
            pm = (m + 3) % 4
            pb, ps = gbufs[pm], gsems[pm]
            if i == 0:

                @pl.when(nb > 0)
                def _(pm=pm):
                    pltpu.make_async_copy(
                        gbufs[pm], acc.at[pl.ds(row0, K)],
                        rsems[pm]).wait()
            else:
                pltpu.make_async_copy(gbufs[pm], acc.at[pl.ds(row0, K)],
                                      rsems[pm]).wait()
            if i < IB - 3:
                pltpu.async_copy(hs_hbm.at[cur.at[i + 3, 0]], pb, ps)
            elif i == IB - 3:

                @pl.when(nb + 1 < NB)
                def _(pb=pb, ps=ps):
                    pltpu.make_async_copy(
                        ed_hbm.at[w, pl.ds((nb + 1) * IB, IB)], nxt,
                        isems[(v + 1) % 2]).wait()
                    pltpu.async_copy(hs_hbm.at[nxt.at[0, 0]], pb, ps)
            else:

                @pl.when(nb + 1 < NB)
                def _(i=i, pb=pb, ps=ps):
                    pltpu.async_copy(hs_hbm.at[nxt.at[i - (IB - 3), 0]],
                                     pb, ps)
            pltpu.make_async_copy(hs_hbm.at[cur.at[i, 0]], gb, gs).wait()
            pltpu.async_copy(gb, acc.at[cur.at[i, 1]], rsems[m], add=True)

        @pl.when(nb + 2 < NB)
        def _():
            pltpu.async_copy(ed_hbm.at[w, pl.ds((nb + 2) * IB, IB)], cur,
                             isems[v % 2])

    def step(nb, c):
        for v in range(4):

            @pl.when(nb % 4 == v)
            def _(v=v):
                block(nb, v)
        return c

    lax.fori_loop(0, NB, step, 0)
    pltpu.make_async_copy(gbufs[(NB * IB - 1) % 4], acc.at[pl.ds(row0, K)],
                          rsems[(NB * IB - 1) % 4]).wait()
    plsc.subcore_barrier()
    pltpu.sync_copy(acc.at[pl.ds(row0, RPT)], out_hbm.at[w])


def _dis(deg_blk):
    return lax.rsqrt(1.0 + deg_blk[:, 0] + deg_blk[:, 1])


def _tc_in_body(x_ref, w_ref, deg_ref, o_ref):
    dis = _dis(deg_ref[...])
    h = jnp.dot(x_ref[...], w_ref[...], precision=lax.Precision.HIGHEST,
                preferred_element_type=jnp.float32)
    o_ref[...] = h * dis[:, None]


def _tc_mid_body(p_ref, hs_ref, deg_ref, b_ref, w_ref, o_ref):
    dis = _dis(deg_ref[...])
    u = (p_ref[0] + p_ref[1] + hs_ref[...]) * dis[:, None] + b_ref[...]
    u = jnp.maximum(u, 0.0)
    h = jnp.dot(u, w_ref[...], precision=lax.Precision.HIGHEST,
                preferred_element_type=jnp.float32)
    o_ref[...] = h * dis[:, None]


def _tc_out_body(p_ref, hs_ref, deg_ref, b_ref, o_ref):
    dis = _dis(deg_ref[...])
    o_ref[...] = (p_ref[0] + p_ref[1] + hs_ref[...]) * dis[:, None] + b_ref[...]


_row_spec = pl.BlockSpec((BR, F), lambda i: (i, 0))
_p_spec = pl.BlockSpec((NC, BR, F), lambda i: (0, i, 0))
_deg_spec = pl.BlockSpec((BR, 2), lambda i: (i, 0))
_w_spec = pl.BlockSpec((F, F), lambda i: (0, 0))
_b_spec = pl.BlockSpec((1, F), lambda i: (0, 0))
_out_sds = jax.ShapeDtypeStruct((N, F), jnp.float32)

_tc_in = pl.pallas_call(
    _tc_in_body, grid=(N // BR,),
    in_specs=[_row_spec, _w_spec, _deg_spec],
    out_specs=_row_spec, out_shape=_out_sds)

_tc_mid = pl.pallas_call(
    _tc_mid_body, grid=(N // BR,),
    in_specs=[_p_spec, _row_spec, _deg_spec, _b_spec, _w_spec],
    out_specs=_row_spec, out_shape=_out_sds)

_tc_out = pl.pallas_call(
    _tc_out_body, grid=(N // BR,),
    in_specs=[_p_spec, _row_spec, _deg_spec, _b_spec],
    out_specs=_row_spec, out_shape=_out_sds)


def kernel(x, edge_index, W1, b1, W2, b2, W3, b3):
    ei = edge_index.astype(jnp.int32)
    ed = jnp.stack(
        [ei[0].reshape(NW, CH, K), ei[1].reshape(NW, CH, K)], axis=2)
    deg2 = _sc_deg(ed).reshape(NC, DEG_PAD)
    degT = deg2[:, :N].T
    b1r, b2r, b3r = (b.reshape(1, F) for b in (b1, b2, b3))

    hs1 = _tc_in(x, W1, degT)
    p1 = _sc_edges(hs1, ed).reshape(NC, N, F)
    hs2 = _tc_mid(p1, hs1, degT, b1r, W2)
    p2 = _sc_edges(hs2, ed).reshape(NC, N, F)
    hs3 = _tc_mid(p2, hs2, degT, b2r, W3)
    p3 = _sc_edges(hs3, ed).reshape(NC, N, F)
    return _tc_out(p3, hs3, degT, b3r)
